# baseline (speedup 1.0000x reference)
import functools

import jax
import jax.numpy as jnp
from jax import lax
from jax.experimental import pallas as pl
from jax.experimental.pallas import tpu as pltpu
from jax.experimental.pallas import tpu_sc as plsc

N = 8192
KNN = 20
NSEG = 8
R = 256
C = 2048
NB = N // R
NT = N // C
INF = float("inf")
BIG = 2**30

_f32 = jnp.float32


def _knn_body(t0_ref, cnt_ref, xp_ref, brow_ref, bcol_ref, wa_ref, ba_ref,
              idx_ref, ua_ref, d2_ref, *, dp, f):
    i = pl.program_id(0)
    r0 = pl.multiple_of(i * R, R)
    xr = xp_ref[pl.ds(r0, R), :]
    ua_ref[...] = (
        jax.lax.dot_general(xr, wa_ref[...], (((1,), (0,)), ((), ())),
                            preferred_element_type=_f32) + ba_ref[...])
    sqr = jnp.sum(xr * xr, axis=1, keepdims=True)
    br = brow_ref[...]
    t0 = t0_ref[i]
    nt = cnt_ref[i]
    ones8 = jnp.ones((8, dp), _f32)
    iota_c = lax.broadcasted_iota(jnp.int32, (R, C), 1).astype(_f32)
    lane16 = lax.broadcasted_iota(jnp.int32, (R, NT), 1).astype(_f32)
    BIGF = jnp.float32(3.0e7)

    def p1(t, cache):
        c0 = pl.multiple_of((t0 + t) * C, C)
        xc = xp_ref[pl.ds(c0, C), :]
        prod = lax.dot_general(xr, xc, (((1,), (1,)), ((), ())),
                               preferred_element_type=_f32)
        sqc = lax.dot_general(ones8, xc * xc, (((1,), (1,)), ((), ())),
                              preferred_element_type=_f32)
        d2 = sqr + sqc[0:1, :] - 2.0 * prod
        bc = bcol_ref[t0 + t]
        d2 = jnp.where(br != bc, INF, d2)
        d2_ref[t] = d2
        tmin = jnp.min(d2, axis=1, keepdims=True)
        return jnp.where(lane16 == t.astype(_f32), tmin, cache)

    cache = lax.fori_loop(0, nt, p1, jnp.full((R, NT), INF))

    cols = []
    for _k in range(KNN):
        m = jnp.min(cache, axis=1, keepdims=True)
        tsel = jnp.min(jnp.where(cache == m, lane16, BIGF), axis=1,
                       keepdims=True)

        def p2(t, carry, m=m, tsel=tsel):
            gi, cache = carry
            tf = t.astype(_f32)
            tile = d2_ref[t]
            hit_row = tsel == tf
            li = jnp.min(jnp.where(tile == m, iota_c, BIGF), axis=1,
                         keepdims=True)
            base = ((t0 + t) * C).astype(_f32)
            gi = jnp.where(hit_row, li + base, gi)
            m2 = jnp.min(jnp.where(tile <= m, INF, tile), axis=1,
                         keepdims=True)
            cache = jnp.where((lane16 == tf) & hit_row, m2, cache)
            return gi, cache

        gi, cache = lax.fori_loop(
            0, nt, p2, (jnp.zeros((R, 1), _f32), cache))
        cols.append(gi)
    gidx = jnp.concatenate(cols, axis=1).astype(jnp.int32)
    idx_ref[...] = jnp.clip(gidx, 0, N - 1)


def _knn(xp, brow, bcol3, wa, ba, t0, cnt, dp, f):
    grid_spec = pltpu.PrefetchScalarGridSpec(
        num_scalar_prefetch=2,
        grid=(NB,),
        in_specs=[
            pl.BlockSpec((N, dp), lambda i, s0, s1: (0, 0)),
            pl.BlockSpec((R, 1), lambda i, s0, s1: (i, 0)),
            pl.BlockSpec((NT, 1, C), lambda i, s0, s1: (0, 0, 0)),
            pl.BlockSpec((dp, f), lambda i, s0, s1: (0, 0)),
            pl.BlockSpec((1, f), lambda i, s0, s1: (0, 0)),
        ],
        out_specs=[
            pl.BlockSpec((R, KNN), lambda i, s0, s1: (i, 0)),
            pl.BlockSpec((R, f), lambda i, s0, s1: (i, 0)),
        ],
        scratch_shapes=[pltpu.VMEM((NT, R, C), _f32)],
    )
    return pl.pallas_call(
        functools.partial(_knn_body, dp=dp, f=f),
        grid_spec=grid_spec,
        compiler_params=pltpu.CompilerParams(
            dimension_semantics=("parallel",)),
        out_shape=[
            jax.ShapeDtypeStruct((N, KNN), jnp.int32),
            jax.ShapeDtypeStruct((N, f), _f32),
        ],
    )(t0, cnt, xp, brow, bcol3, wa, ba)


def _sc_gather(table, idx_flat):
    d = table.shape[1]
    nk = idx_flat.shape[0]
    nw = 32
    bpw = nk // nw
    ch = 1024
    nch = bpw // ch
    mesh = plsc.VectorSubcoreMesh(core_axis_name="c", subcore_axis_name="s")

    @functools.partial(
        pl.kernel, mesh=mesh,
        out_type=jax.ShapeDtypeStruct((nk, d), _f32),
        compiler_params=pltpu.CompilerParams(use_tc_tiling_on_sc=False),
        scratch_types=[
            pltpu.VMEM((ch,), jnp.int32),
            pltpu.VMEM((ch, d), _f32),
            pltpu.SemaphoreType.DMA,
        ])
    def k(table_hbm, idx_hbm, out_hbm, idx_v, rows_v, sem):
        wid = lax.axis_index("s") * 2 + lax.axis_index("c")
        base = wid * bpw

        @pl.loop(0, nch)
        def _(ci):
            off = base + ci * ch
            pltpu.sync_copy(idx_hbm.at[pl.ds(off, ch)], idx_v)
            pltpu.async_copy(table_hbm.at[idx_v], rows_v, sem).wait()
            pltpu.sync_copy(rows_v, out_hbm.at[pl.ds(off, ch)])

    return k(table, idx_flat)


def _conv1_body(g_ref, u_ref, w1b_ref, w2_ref, b2_ref, w3_ref, b3_ref, h_ref):
    vg = jnp.dot(g_ref[...], w1b_ref[...], preferred_element_type=_f32)
    u3 = jnp.broadcast_to(u_ref[...][:, None, :], (R, KNN, 64))
    e = jnp.maximum(u3.reshape(R * KNN, 64) + vg, 0.0)
    e = jnp.maximum(
        jnp.dot(e, w2_ref[...], preferred_element_type=_f32) + b2_ref[...], 0.0)
    e = jnp.maximum(
        jnp.dot(e, w3_ref[...], preferred_element_type=_f32) + b3_ref[...], 0.0)
    h_ref[...] = jnp.max(e.reshape(R, KNN, 64), axis=1)


def _conv1(g, u, w1b, w2, b2, w3, b3):
    return pl.pallas_call(
        _conv1_body,
        grid=(NB,),
        compiler_params=pltpu.CompilerParams(
            dimension_semantics=("parallel",)),
        in_specs=[
            pl.BlockSpec((R * KNN, 16), lambda i: (i, 0)),
            pl.BlockSpec((R, 64), lambda i: (i, 0)),
            pl.BlockSpec((16, 64), lambda i: (0, 0)),
            pl.BlockSpec((64, 64), lambda i: (0, 0)),
            pl.BlockSpec((1, 64), lambda i: (0, 0)),
            pl.BlockSpec((64, 64), lambda i: (0, 0)),
            pl.BlockSpec((1, 64), lambda i: (0, 0)),
        ],
        out_specs=pl.BlockSpec((R, 64), lambda i: (i, 0)),
        out_shape=jax.ShapeDtypeStruct((N, 64), _f32),
    )(g, u, w1b, w2, b2, w3, b3)


def _conv2_body(g_ref, c_ref, w2b_ref, h_ref):
    dg = jnp.dot(g_ref[...], w2b_ref[...], preferred_element_type=_f32)
    dmax = jnp.max(dg.reshape(R, KNN, 256), axis=1)
    h_ref[...] = jnp.maximum(c_ref[...] + dmax, 0.0)


def _conv2(g, c, w2b):
    return pl.pallas_call(
        _conv2_body,
        grid=(NB,),
        compiler_params=pltpu.CompilerParams(
            dimension_semantics=("parallel",)),
        in_specs=[
            pl.BlockSpec((R * KNN, 64), lambda i: (i, 0)),
            pl.BlockSpec((R, 256), lambda i: (i, 0)),
            pl.BlockSpec((64, 256), lambda i: (0, 0)),
        ],
        out_specs=pl.BlockSpec((R, 256), lambda i: (i, 0)),
        out_shape=jax.ShapeDtypeStruct((N, 256), _f32),
    )(g, c, w2b)


def _head_body(h1_ref, h2_ref, b1_ref, b2_ref, linw_ref, linb_ref,
               l1w_ref, l1b_ref, l2w_ref, l2b_ref,
               mw1_ref, mb1_ref, mw2_ref, mb2_ref, o_ref):
    def pool(href, bref):
        def body(i, acc):
            r0 = pl.multiple_of(i * R, R)
            hb = href[pl.ds(r0, R), :]
            bb = bref[pl.ds(r0, R), :]
            rows = [jnp.max(jnp.where(bb == s, hb, -INF), axis=0,
                            keepdims=True) for s in range(NSEG)]
            return jnp.maximum(acc, jnp.concatenate(rows, axis=0))
        return lax.fori_loop(0, NB, body, jnp.full((NSEG, 256), -INF))

    p1 = pool(h1_ref, b1_ref)
    p2 = pool(h2_ref, b2_ref)
    bo1 = jnp.maximum(
        jnp.dot(p1, linw_ref[...], preferred_element_type=_f32)
        + linb_ref[...], 0.0)
    bo2 = jnp.maximum(
        jnp.dot(p2, linw_ref[...], preferred_element_type=_f32)
        + linb_ref[...], 0.0)
    xd = bo2 - bo1
    xa = jnp.dot(xd, l1w_ref[...], preferred_element_type=_f32) + l1b_ref[...]
    xb = jnp.dot(xd, l2w_ref[...], preferred_element_type=_f32) + l2b_ref[...]
    xo = jnp.maximum(xa - xb, 0.0)
    xo = jnp.maximum(
        jnp.dot(xo, mw1_ref[...], preferred_element_type=_f32) + mb1_ref[...],
        0.0)
    xo = jnp.dot(xo, mw2_ref[...], preferred_element_type=_f32) + mb2_ref[...]
    mx = jnp.max(xo, axis=1, keepdims=True)
    lse = jnp.log(jnp.sum(jnp.exp(xo - mx), axis=1, keepdims=True))
    o_ref[...] = xo - mx - lse


def _head(h1, h2, b1, b2, lin_w, lin_b, l1_w, l1_b, l2_w, l2_b,
          m_w1, m_b1, m_w2, m_b2):
    def full(shape):
        return pl.BlockSpec(shape, lambda: tuple(0 for _ in shape))
    return pl.pallas_call(
        _head_body,
        in_specs=[
            full((N, 256)), full((N, 256)), full((N, 1)), full((N, 1)),
            full((256, 512)), full((1, 512)),
            full((512, 128)), full((1, 128)),
            full((512, 128)), full((1, 128)),
            full((128, 64)), full((1, 64)),
            full((64, 5)), full((1, 5)),
        ],
        out_specs=full((NSEG, 5)),
        out_shape=jax.ShapeDtypeStruct((NSEG, 5), _f32),
    )(h1, h2, b1, b2, lin_w, lin_b, l1_w, l1_b, l2_w, l2_b,
      m_w1, m_b1, m_w2, m_b2)


def _windows(batch):
    segs = jnp.arange(NSEG, dtype=jnp.int32)
    lo = jnp.searchsorted(batch, segs, side="left").astype(jnp.int32)
    hi = jnp.searchsorted(batch, segs, side="right").astype(jnp.int32)
    rb = jnp.arange(NB, dtype=jnp.int32)
    b_first = batch[rb * R]
    b_last = batch[rb * R + (R - 1)]
    col_lo = lo[b_first]
    col_hi = hi[b_last]
    t0 = col_lo // C
    cnt = jnp.maximum((col_hi + C - 1) // C - t0, 1).astype(jnp.int32)
    return t0.astype(jnp.int32), cnt


def _cloud(x, batch, c1_w1, c1_b1, c1_w2, c1_b2, c1_w3, c1_b3, c2_w, c2_b):
    xp = jnp.pad(x, ((0, 0), (0, 16 - x.shape[1])))
    brow = batch[:, None]
    bcol3 = batch.reshape(NT, 1, C)
    t0, cnt = _windows(batch)

    w1a = jnp.pad(c1_w1[:6] - c1_w1[6:], ((0, 10), (0, 0)))
    w1b = jnp.pad(c1_w1[6:], ((0, 10), (0, 0)))
    idx1, u1 = _knn(xp, brow, bcol3, w1a, c1_b1[None, :], t0, cnt, 16, 64)
    g1 = _sc_gather(xp, idx1.reshape(-1))
    h1 = _conv1(g1, u1, w1b, c1_w2, c1_b2[None, :], c1_w3, c1_b3[None, :])

    w2a = c2_w[:64] - c2_w[64:]
    w2b = c2_w[64:]
    idx2, c = _knn(h1, brow, bcol3, w2a, c2_b[None, :], t0, cnt, 64, 256)
    g2 = _sc_gather(h1, idx2.reshape(-1))
    return _conv2(g2, c, w2b)


def kernel(x, x2, batch, batch2, c1_w1, c1_b1, c1_w2, c1_b2, c1_w3, c1_b3,
           c2_w, c2_b, lin_w, lin_b, l1_w, l1_b, l2_w, l2_b,
           m_w1, m_b1, m_w2, m_b2):
    convs = (c1_w1, c1_b1, c1_w2, c1_b2, c1_w3, c1_b3, c2_w, c2_b)
    h1 = _cloud(x, batch, *convs)
    h2 = _cloud(x2, batch2, *convs)
    return _head(h1, h2, batch[:, None], batch2[:, None],
                 lin_w, lin_b[None, :], l1_w, l1_b[None, :],
                 l2_w, l2_b[None, :], m_w1, m_b1[None, :],
                 m_w2, m_b2[None, :])


# speedup vs baseline: 1.2002x; 1.2002x over previous
import functools

import jax
import jax.numpy as jnp
from jax import lax
from jax.experimental import pallas as pl
from jax.experimental.pallas import tpu as pltpu
from jax.experimental.pallas import tpu_sc as plsc

N = 8192
KNN = 20
NSEG = 8
R = 512
C = 1024
NB = N // R
NT = N // C
INF = float("inf")
BIG = 2**30

_f32 = jnp.float32


def _knn_body(t0_ref, cnt_ref, xp_ref, brow_ref, bcol_ref, wa_ref, ba_ref,
              idx_ref, ua_ref, d2_ref, *, dp, f):
    i = pl.program_id(0)
    r0 = pl.multiple_of(i * R, R)
    xr = xp_ref[pl.ds(r0, R), :]
    ua_ref[...] = (
        jax.lax.dot_general(xr, wa_ref[...], (((1,), (0,)), ((), ())),
                            preferred_element_type=_f32) + ba_ref[...])
    sqr = jnp.sum(xr * xr, axis=1, keepdims=True)
    br = brow_ref[...]
    t0 = t0_ref[i]
    nt = cnt_ref[i]
    ones8 = jnp.ones((8, dp), _f32)
    iota_c = lax.broadcasted_iota(jnp.int32, (R, C), 1).astype(_f32)
    lane16 = lax.broadcasted_iota(jnp.int32, (R, NT), 1).astype(_f32)
    BIGF = jnp.float32(3.0e7)

    def p1(t, cache):
        c0 = pl.multiple_of((t0 + t) * C, C)
        xc = xp_ref[pl.ds(c0, C), :]
        prod = lax.dot_general(xr, xc, (((1,), (1,)), ((), ())),
                               preferred_element_type=_f32)
        sqc = lax.dot_general(ones8, xc * xc, (((1,), (1,)), ((), ())),
                              preferred_element_type=_f32)
        d2 = sqr + sqc[0:1, :] - 2.0 * prod
        bc = bcol_ref[t0 + t]
        d2 = jnp.where(br != bc, INF, d2)
        d2_ref[t] = d2
        tmin = jnp.min(d2, axis=1, keepdims=True)
        return jnp.where(lane16 == t.astype(_f32), tmin, cache)

    cache = lax.fori_loop(0, nt, p1, jnp.full((R, NT), INF))

    cols = []
    for _k in range(KNN):
        m = jnp.min(cache, axis=1, keepdims=True)
        tsel = jnp.min(jnp.where(cache == m, lane16, BIGF), axis=1,
                       keepdims=True)

        def p2(t, carry, m=m, tsel=tsel):
            gi, cache = carry
            tf = t.astype(_f32)
            tile = d2_ref[t]
            hit_row = tsel == tf
            li = jnp.min(jnp.where(tile == m, iota_c, BIGF), axis=1,
                         keepdims=True)
            base = ((t0 + t) * C).astype(_f32)
            gi = jnp.where(hit_row, li + base, gi)
            m2 = jnp.min(jnp.where(tile <= m, INF, tile), axis=1,
                         keepdims=True)
            cache = jnp.where((lane16 == tf) & hit_row, m2, cache)
            return gi, cache

        gi, cache = lax.fori_loop(
            0, nt, p2, (jnp.zeros((R, 1), _f32), cache))
        cols.append(gi)
    gidx = jnp.concatenate(cols, axis=1).astype(jnp.int32)
    idx_ref[...] = jnp.clip(gidx, 0, N - 1)


def _knn(xp, brow, bcol3, wa, ba, t0, cnt, dp, f):
    grid_spec = pltpu.PrefetchScalarGridSpec(
        num_scalar_prefetch=2,
        grid=(NB,),
        in_specs=[
            pl.BlockSpec((N, dp), lambda i, s0, s1: (0, 0)),
            pl.BlockSpec((R, 1), lambda i, s0, s1: (i, 0)),
            pl.BlockSpec((NT, 1, C), lambda i, s0, s1: (0, 0, 0)),
            pl.BlockSpec((dp, f), lambda i, s0, s1: (0, 0)),
            pl.BlockSpec((1, f), lambda i, s0, s1: (0, 0)),
        ],
        out_specs=[
            pl.BlockSpec((R, KNN), lambda i, s0, s1: (i, 0)),
            pl.BlockSpec((R, f), lambda i, s0, s1: (i, 0)),
        ],
        scratch_shapes=[pltpu.VMEM((NT, R, C), _f32)],
    )
    return pl.pallas_call(
        functools.partial(_knn_body, dp=dp, f=f),
        grid_spec=grid_spec,
        compiler_params=pltpu.CompilerParams(
            dimension_semantics=("parallel",)),
        out_shape=[
            jax.ShapeDtypeStruct((N, KNN), jnp.int32),
            jax.ShapeDtypeStruct((N, f), _f32),
        ],
    )(t0, cnt, xp, brow, bcol3, wa, ba)


def _sc_gather(table, idx_flat):
    d = table.shape[1]
    nk = idx_flat.shape[0]
    nw = 32
    bpw = nk // nw
    ch = 1024
    nch = bpw // ch
    mesh = plsc.VectorSubcoreMesh(core_axis_name="c", subcore_axis_name="s")

    @functools.partial(
        pl.kernel, mesh=mesh,
        out_type=jax.ShapeDtypeStruct((nk, d), _f32),
        compiler_params=pltpu.CompilerParams(use_tc_tiling_on_sc=False),
        scratch_types=[
            pltpu.VMEM((ch,), jnp.int32),
            pltpu.VMEM((ch, d), _f32),
            pltpu.SemaphoreType.DMA,
        ])
    def k(table_hbm, idx_hbm, out_hbm, idx_v, rows_v, sem):
        wid = lax.axis_index("s") * 2 + lax.axis_index("c")
        base = wid * bpw

        @pl.loop(0, nch)
        def _(ci):
            off = base + ci * ch
            pltpu.sync_copy(idx_hbm.at[pl.ds(off, ch)], idx_v)
            pltpu.async_copy(table_hbm.at[idx_v], rows_v, sem).wait()
            pltpu.sync_copy(rows_v, out_hbm.at[pl.ds(off, ch)])

    return k(table, idx_flat)


def _conv1_body(g_ref, u_ref, w1b_ref, w2_ref, b2_ref, w3_ref, b3_ref, h_ref):
    vg = jnp.dot(g_ref[...], w1b_ref[...], preferred_element_type=_f32)
    u3 = jnp.broadcast_to(u_ref[...][:, None, :], (R, KNN, 64))
    e = jnp.maximum(u3.reshape(R * KNN, 64) + vg, 0.0)
    e = jnp.maximum(
        jnp.dot(e, w2_ref[...], preferred_element_type=_f32) + b2_ref[...], 0.0)
    e = jnp.maximum(
        jnp.dot(e, w3_ref[...], preferred_element_type=_f32) + b3_ref[...], 0.0)
    h_ref[...] = jnp.max(e.reshape(R, KNN, 64), axis=1)


def _conv1(g, u, w1b, w2, b2, w3, b3):
    return pl.pallas_call(
        _conv1_body,
        grid=(NB,),
        compiler_params=pltpu.CompilerParams(
            dimension_semantics=("parallel",)),
        in_specs=[
            pl.BlockSpec((R * KNN, 16), lambda i: (i, 0)),
            pl.BlockSpec((R, 64), lambda i: (i, 0)),
            pl.BlockSpec((16, 64), lambda i: (0, 0)),
            pl.BlockSpec((64, 64), lambda i: (0, 0)),
            pl.BlockSpec((1, 64), lambda i: (0, 0)),
            pl.BlockSpec((64, 64), lambda i: (0, 0)),
            pl.BlockSpec((1, 64), lambda i: (0, 0)),
        ],
        out_specs=pl.BlockSpec((R, 64), lambda i: (i, 0)),
        out_shape=jax.ShapeDtypeStruct((N, 64), _f32),
    )(g, u, w1b, w2, b2, w3, b3)


def _conv2_body(g_ref, c_ref, w2b_ref, h_ref):
    dg = jnp.dot(g_ref[...], w2b_ref[...], preferred_element_type=_f32)
    dmax = jnp.max(dg.reshape(R, KNN, 256), axis=1)
    h_ref[...] = jnp.maximum(c_ref[...] + dmax, 0.0)


def _conv2(g, c, w2b):
    return pl.pallas_call(
        _conv2_body,
        grid=(NB,),
        compiler_params=pltpu.CompilerParams(
            dimension_semantics=("parallel",)),
        in_specs=[
            pl.BlockSpec((R * KNN, 64), lambda i: (i, 0)),
            pl.BlockSpec((R, 256), lambda i: (i, 0)),
            pl.BlockSpec((64, 256), lambda i: (0, 0)),
        ],
        out_specs=pl.BlockSpec((R, 256), lambda i: (i, 0)),
        out_shape=jax.ShapeDtypeStruct((N, 256), _f32),
    )(g, c, w2b)


def _head_body(h1_ref, h2_ref, b1_ref, b2_ref, linw_ref, linb_ref,
               l1w_ref, l1b_ref, l2w_ref, l2b_ref,
               mw1_ref, mb1_ref, mw2_ref, mb2_ref, o_ref):
    def pool(href, bref):
        def body(i, acc):
            r0 = pl.multiple_of(i * R, R)
            hb = href[pl.ds(r0, R), :]
            bb = bref[pl.ds(r0, R), :]
            rows = [jnp.max(jnp.where(bb == s, hb, -INF), axis=0,
                            keepdims=True) for s in range(NSEG)]
            return jnp.maximum(acc, jnp.concatenate(rows, axis=0))
        return lax.fori_loop(0, NB, body, jnp.full((NSEG, 256), -INF))

    p1 = pool(h1_ref, b1_ref)
    p2 = pool(h2_ref, b2_ref)
    bo1 = jnp.maximum(
        jnp.dot(p1, linw_ref[...], preferred_element_type=_f32)
        + linb_ref[...], 0.0)
    bo2 = jnp.maximum(
        jnp.dot(p2, linw_ref[...], preferred_element_type=_f32)
        + linb_ref[...], 0.0)
    xd = bo2 - bo1
    xa = jnp.dot(xd, l1w_ref[...], preferred_element_type=_f32) + l1b_ref[...]
    xb = jnp.dot(xd, l2w_ref[...], preferred_element_type=_f32) + l2b_ref[...]
    xo = jnp.maximum(xa - xb, 0.0)
    xo = jnp.maximum(
        jnp.dot(xo, mw1_ref[...], preferred_element_type=_f32) + mb1_ref[...],
        0.0)
    xo = jnp.dot(xo, mw2_ref[...], preferred_element_type=_f32) + mb2_ref[...]
    mx = jnp.max(xo, axis=1, keepdims=True)
    lse = jnp.log(jnp.sum(jnp.exp(xo - mx), axis=1, keepdims=True))
    o_ref[...] = xo - mx - lse


def _head(h1, h2, b1, b2, lin_w, lin_b, l1_w, l1_b, l2_w, l2_b,
          m_w1, m_b1, m_w2, m_b2):
    def full(shape):
        return pl.BlockSpec(shape, lambda: tuple(0 for _ in shape))
    return pl.pallas_call(
        _head_body,
        in_specs=[
            full((N, 256)), full((N, 256)), full((N, 1)), full((N, 1)),
            full((256, 512)), full((1, 512)),
            full((512, 128)), full((1, 128)),
            full((512, 128)), full((1, 128)),
            full((128, 64)), full((1, 64)),
            full((64, 5)), full((1, 5)),
        ],
        out_specs=full((NSEG, 5)),
        out_shape=jax.ShapeDtypeStruct((NSEG, 5), _f32),
    )(h1, h2, b1, b2, lin_w, lin_b, l1_w, l1_b, l2_w, l2_b,
      m_w1, m_b1, m_w2, m_b2)


def _windows(batch):
    segs = jnp.arange(NSEG, dtype=jnp.int32)
    lo = jnp.searchsorted(batch, segs, side="left").astype(jnp.int32)
    hi = jnp.searchsorted(batch, segs, side="right").astype(jnp.int32)
    rb = jnp.arange(NB, dtype=jnp.int32)
    b_first = batch[rb * R]
    b_last = batch[rb * R + (R - 1)]
    col_lo = lo[b_first]
    col_hi = hi[b_last]
    t0 = col_lo // C
    cnt = jnp.maximum((col_hi + C - 1) // C - t0, 1).astype(jnp.int32)
    return t0.astype(jnp.int32), cnt


def _cloud(x, batch, c1_w1, c1_b1, c1_w2, c1_b2, c1_w3, c1_b3, c2_w, c2_b):
    xp = jnp.pad(x, ((0, 0), (0, 16 - x.shape[1])))
    brow = batch[:, None]
    bcol3 = batch.reshape(NT, 1, C)
    t0, cnt = _windows(batch)

    w1a = jnp.pad(c1_w1[:6] - c1_w1[6:], ((0, 10), (0, 0)))
    w1b = jnp.pad(c1_w1[6:], ((0, 10), (0, 0)))
    idx1, u1 = _knn(xp, brow, bcol3, w1a, c1_b1[None, :], t0, cnt, 16, 64)
    g1 = _sc_gather(xp, idx1.reshape(-1))
    h1 = _conv1(g1, u1, w1b, c1_w2, c1_b2[None, :], c1_w3, c1_b3[None, :])

    w2a = c2_w[:64] - c2_w[64:]
    w2b = c2_w[64:]
    idx2, c = _knn(h1, brow, bcol3, w2a, c2_b[None, :], t0, cnt, 64, 256)
    g2 = _sc_gather(h1, idx2.reshape(-1))
    return _conv2(g2, c, w2b)


def kernel(x, x2, batch, batch2, c1_w1, c1_b1, c1_w2, c1_b2, c1_w3, c1_b3,
           c2_w, c2_b, lin_w, lin_b, l1_w, l1_b, l2_w, l2_b,
           m_w1, m_b1, m_w2, m_b2):
    convs = (c1_w1, c1_b1, c1_w2, c1_b2, c1_w3, c1_b3, c2_w, c2_b)
    h1 = _cloud(x, batch, *convs)
    h2 = _cloud(x2, batch2, *convs)
    return _head(h1, h2, batch[:, None], batch2[:, None],
                 lin_w, lin_b[None, :], l1_w, l1_b[None, :],
                 l2_w, l2_b[None, :], m_w1, m_b1[None, :],
                 m_w2, m_b2[None, :])
